# Initial kernel scaffold; baseline (speedup 1.0000x reference)
import functools

import jax
import jax.numpy as jnp
from jax import lax
from jax.experimental import pallas as pl
from jax.experimental.pallas import tpu as pltpu
from jax.experimental.pallas import tpu_sc as plsc

N = 10000
D = 128
F = 16
E = 320000
B = 4096
NF = 26
NC, NS = 2, 16
NW = NC * NS
NPAD = 10240
DEG_SLICE = NPAD // NS
ROWS_PER_TILE = N // NS
CHUNK = 100
CHUNKS_PER_TILE = E // NW // CHUNK
GROUP = 10
XCHUNK = 128
XCH_PER_TILE = B * NF // NW // XCHUNK
XROWS = B * NF // NW

_MESH = plsc.VectorSubcoreMesh(
    core_axis_name="c", subcore_axis_name="s", num_cores=NC, num_subcores=NS)


@functools.partial(
    pl.kernel,
    out_type=jax.ShapeDtypeStruct((NC, NPAD), jnp.float32),
    mesh=_MESH,
    scratch_types=[
        pltpu.VMEM((CHUNKS_PER_TILE, CHUNK), jnp.int32),
        pltpu.VMEM((128,), jnp.float32),
        pltpu.VMEM((DEG_SLICE,), jnp.float32),
        pltpu.VMEM_SHARED((NPAD,), jnp.float32),
        pltpu.SemaphoreType.DMA,
    ],
)
def _deg_kernel(dst_hbm, deg_hbm, didx, ones, zbuf, deg_sh, sem):
    cid = lax.axis_index("c")
    sid = lax.axis_index("s")
    wid = cid * NS + sid
    for i in range(128 // F):
        ones[pl.ds(i * F, F)] = jnp.ones((F,), jnp.float32)
    for i in range(DEG_SLICE // F):
        zbuf[pl.ds(i * F, F)] = jnp.zeros((F,), jnp.float32)
    pltpu.sync_copy(zbuf, deg_sh.at[pl.ds(sid * DEG_SLICE, DEG_SLICE)])
    plsc.subcore_barrier()
    pltpu.sync_copy(dst_hbm.at[pl.ds(wid * CHUNKS_PER_TILE, CHUNKS_PER_TILE)], didx)

    def group_body(g, carry):
        j0 = g * GROUP
        descs = []
        for i in range(GROUP):
            descs.append(pltpu.async_copy(
                ones.at[pl.ds(0, CHUNK)], deg_sh.at[didx.at[j0 + i]], sem,
                add=True))
        for d in descs:
            d.wait()
        return carry

    lax.fori_loop(0, CHUNKS_PER_TILE // GROUP, group_body, 0)
    plsc.subcore_barrier()
    pltpu.sync_copy(deg_sh.at[pl.ds(sid * DEG_SLICE, DEG_SLICE)],
                    deg_hbm.at[cid, pl.ds(sid * DEG_SLICE, DEG_SLICE)])


_BLK = 1000


def _mm_body(degT_ref, feat_ref, w_ref, g_ref, dinv_ref):
    deg = jnp.sum(degT_ref[...], axis=1, keepdims=True) + 1.0
    dinv = lax.rsqrt(deg)
    h = jnp.dot(feat_ref[...], w_ref[...], preferred_element_type=jnp.float32)
    g_ref[...] = h * dinv
    dinv_ref[...] = jnp.broadcast_to(dinv, (_BLK, F))


_mm_call = pl.pallas_call(
    _mm_body,
    grid=(N // _BLK,),
    in_specs=[
        pl.BlockSpec((_BLK, NC), lambda i: (i, 0)),
        pl.BlockSpec((_BLK, D), lambda i: (i, 0)),
        pl.BlockSpec((D, F), lambda i: (0, 0)),
    ],
    out_specs=[
        pl.BlockSpec((_BLK, F), lambda i: (i, 0)),
        pl.BlockSpec((_BLK, F), lambda i: (i, 0)),
    ],
    out_shape=[
        jax.ShapeDtypeStruct((N, F), jnp.float32),
        jax.ShapeDtypeStruct((N, F), jnp.float32),
    ],
)


@functools.partial(
    pl.kernel,
    out_type=jax.ShapeDtypeStruct((NC, N, F), jnp.float32),
    mesh=_MESH,
    scratch_types=[
        pltpu.VMEM((CHUNKS_PER_TILE, CHUNK), jnp.int32),
        pltpu.VMEM((CHUNKS_PER_TILE, CHUNK), jnp.int32),
        pltpu.VMEM((GROUP, CHUNK, F), jnp.float32),
        pltpu.VMEM_SHARED((N, F), jnp.float32),
        pltpu.SemaphoreType.DMA,
        pltpu.SemaphoreType.DMA,
    ],
)
def _agg_kernel(g_hbm, src_hbm, dst_hbm, acc_hbm, sidx, didx, rows, acc_sh,
                gsem, ssem):
    cid = lax.axis_index("c")
    sid = lax.axis_index("s")
    wid = cid * NS + sid
    pltpu.sync_copy(g_hbm.at[pl.ds(sid * ROWS_PER_TILE, ROWS_PER_TILE)],
                    acc_sh.at[pl.ds(sid * ROWS_PER_TILE, ROWS_PER_TILE)])
    plsc.subcore_barrier()
    pltpu.sync_copy(src_hbm.at[pl.ds(wid * CHUNKS_PER_TILE, CHUNKS_PER_TILE)], sidx)
    pltpu.sync_copy(dst_hbm.at[pl.ds(wid * CHUNKS_PER_TILE, CHUNKS_PER_TILE)], didx)

    def group_body(g, carry):
        j0 = g * GROUP
        gd = []
        for i in range(GROUP):
            gd.append(pltpu.async_copy(
                g_hbm.at[sidx.at[j0 + i]], rows.at[i], gsem))
        for d in gd:
            d.wait()
        sd = []
        for i in range(GROUP):
            sd.append(pltpu.async_copy(
                rows.at[i], acc_sh.at[didx.at[j0 + i]], ssem, add=True))
        for d in sd:
            d.wait()
        return carry

    lax.fori_loop(0, CHUNKS_PER_TILE // GROUP, group_body, 0)
    plsc.subcore_barrier()
    pltpu.sync_copy(acc_sh.at[pl.ds(sid * ROWS_PER_TILE, ROWS_PER_TILE)],
                    acc_hbm.at[cid, pl.ds(sid * ROWS_PER_TILE, ROWS_PER_TILE)])


def _fin_body(acc_ref, g_ref, dinv_ref, b_ref, out_ref):
    out_ref[...] = (dinv_ref[...] * (acc_ref[0] + acc_ref[1] - g_ref[...])
                    + b_ref[...])


_fin_call = pl.pallas_call(
    _fin_body,
    grid=(N // _BLK,),
    in_specs=[
        pl.BlockSpec((NC, _BLK, F), lambda i: (0, i, 0)),
        pl.BlockSpec((_BLK, F), lambda i: (i, 0)),
        pl.BlockSpec((_BLK, F), lambda i: (i, 0)),
        pl.BlockSpec((1, F), lambda i: (0, 0)),
    ],
    out_specs=pl.BlockSpec((_BLK, F), lambda i: (i, 0)),
    out_shape=jax.ShapeDtypeStruct((N, F), jnp.float32),
)


@functools.partial(
    pl.kernel,
    out_type=jax.ShapeDtypeStruct((B * NF, F), jnp.float32),
    mesh=_MESH,
    scratch_types=[
        pltpu.VMEM((XCH_PER_TILE, XCHUNK), jnp.int32),
        pltpu.VMEM((XROWS, F), jnp.float32),
        pltpu.SemaphoreType.DMA,
    ],
)
def _gather_kernel(out_hbm, x_hbm, y_hbm, xidx, rows, sem):
    cid = lax.axis_index("c")
    sid = lax.axis_index("s")
    wid = cid * NS + sid
    pltpu.sync_copy(x_hbm.at[wid], xidx)
    for h in range(2):
        descs = []
        for i in range(XCH_PER_TILE // 2):
            j = h * (XCH_PER_TILE // 2) + i
            descs.append(pltpu.async_copy(
                out_hbm.at[xidx.at[j]], rows.at[pl.ds(j * XCHUNK, XCHUNK)],
                sem))
        for d in descs:
            d.wait()
    pltpu.sync_copy(rows, y_hbm.at[pl.ds(wid * XROWS, XROWS)])


@jax.jit
def _run(features, train_mat, W, b, x):
    srcr = train_mat[0].reshape(E // CHUNK, CHUNK)
    dstr = train_mat[1].reshape(E // CHUNK, CHUNK)
    deg_parts = _deg_kernel(dstr)
    degT = deg_parts.T[:N]
    g, dinv16 = _mm_call(degT, features, W)
    acc_parts = _agg_kernel(g, srcr, dstr)
    out = _fin_call(acc_parts, g, dinv16, b.reshape(1, F))
    xr = x.reshape(NW, XCH_PER_TILE, XCHUNK)
    y = _gather_kernel(out, xr)
    return y.reshape(B, NF, F)


def kernel(features, train_mat, W, b, x):
    return _run(features, train_mat, W, b, x)


# speedup vs baseline: 29.7248x; 29.7248x over previous
import functools

import jax
import jax.numpy as jnp
from jax import lax
from jax.experimental import pallas as pl
from jax.experimental.pallas import tpu as pltpu
from jax.experimental.pallas import tpu_sc as plsc

N = 10000
D = 128
F = 16
E = 320000
B = 4096
NF = 26
NC, NS = 2, 16
NW = NC * NS
NPAD = 10240
NSLICE = NPAD // NS
CHUNK = 125
CHUNKS_PER_TILE = E // NW // CHUNK
GROUP = 10
XCHUNK = 128
XCH_PER_TILE = B * NF // NW // XCHUNK
XROWS = B * NF // NW

_MESH = plsc.VectorSubcoreMesh(
    core_axis_name="c", subcore_axis_name="s", num_cores=NC, num_subcores=NS)


@functools.partial(
    pl.kernel,
    out_type=jax.ShapeDtypeStruct((NC * NPAD,), jnp.float32),
    mesh=_MESH,
    compiler_params=pltpu.CompilerParams(use_tc_tiling_on_sc=False),
    scratch_types=[
        pltpu.VMEM((CHUNKS_PER_TILE, CHUNK), jnp.int32),
        pltpu.VMEM((128,), jnp.float32),
        pltpu.VMEM((NSLICE,), jnp.float32),
        pltpu.VMEM_SHARED((NPAD,), jnp.float32),
        pltpu.SemaphoreType.DMA,
    ],
)
def _deg_kernel(dst_hbm, deg_hbm, didx, ones, zbuf, deg_sh, sem):
    cid = lax.axis_index("c")
    sid = lax.axis_index("s")
    wid = cid * NS + sid
    for i in range(128 // F):
        ones[pl.ds(i * F, F)] = jnp.ones((F,), jnp.float32)
    for i in range(NSLICE // F):
        zbuf[pl.ds(i * F, F)] = jnp.zeros((F,), jnp.float32)
    pltpu.sync_copy(zbuf, deg_sh.at[pl.ds(sid * NSLICE, NSLICE)])
    plsc.subcore_barrier()
    pltpu.sync_copy(dst_hbm.at[pl.ds(wid * CHUNKS_PER_TILE, CHUNKS_PER_TILE)], didx)

    def group_body(g, carry):
        j0 = g * GROUP
        descs = []
        for i in range(GROUP):
            descs.append(pltpu.async_copy(
                ones.at[pl.ds(0, CHUNK)], deg_sh.at[didx.at[j0 + i]], sem,
                add=True))
        for d in descs:
            d.wait()
        return carry

    lax.fori_loop(0, CHUNKS_PER_TILE // GROUP, group_body, 0)
    plsc.subcore_barrier()
    pltpu.sync_copy(deg_sh.at[pl.ds(sid * NSLICE, NSLICE)],
                    deg_hbm.at[pl.ds(cid * NPAD + sid * NSLICE, NSLICE)])


_MMBLK = 1024


def _mm_body(degT_ref, feat_ref, w_ref, g_ref, dinv_ref):
    deg = jnp.sum(degT_ref[...], axis=1, keepdims=True) + 1.0
    dinv = lax.rsqrt(deg)
    h = jnp.dot(feat_ref[...], w_ref[...], preferred_element_type=jnp.float32)
    g_ref[...] = h * dinv
    dinv_ref[...] = jnp.broadcast_to(dinv, (_MMBLK, F))


_mm_call = pl.pallas_call(
    _mm_body,
    grid=(NPAD // _MMBLK,),
    in_specs=[
        pl.BlockSpec((_MMBLK, NC), lambda i: (i, 0)),
        pl.BlockSpec((_MMBLK, D), lambda i: (i, 0)),
        pl.BlockSpec((D, F), lambda i: (0, 0)),
    ],
    out_specs=[
        pl.BlockSpec((_MMBLK, F), lambda i: (i, 0)),
        pl.BlockSpec((_MMBLK, F), lambda i: (i, 0)),
    ],
    out_shape=[
        jax.ShapeDtypeStruct((NPAD, F), jnp.float32),
        jax.ShapeDtypeStruct((NPAD, F), jnp.float32),
    ],
)


@functools.partial(
    pl.kernel,
    out_type=jax.ShapeDtypeStruct((NC, NPAD, F), jnp.float32),
    mesh=_MESH,
    compiler_params=pltpu.CompilerParams(use_tc_tiling_on_sc=False),
    scratch_types=[
        pltpu.VMEM((CHUNKS_PER_TILE, CHUNK), jnp.int32),
        pltpu.VMEM((CHUNKS_PER_TILE, CHUNK), jnp.int32),
        pltpu.VMEM((GROUP, CHUNK, F), jnp.float32),
        pltpu.VMEM_SHARED((NPAD, F), jnp.float32),
        pltpu.SemaphoreType.DMA,
        pltpu.SemaphoreType.DMA,
    ],
)
def _agg_kernel(g_hbm, src_hbm, dst_hbm, acc_hbm, sidx, didx, rows, acc_sh,
                gsem, ssem):
    cid = lax.axis_index("c")
    sid = lax.axis_index("s")
    wid = cid * NS + sid
    pltpu.sync_copy(g_hbm.at[pl.ds(sid * NSLICE, NSLICE)],
                    acc_sh.at[pl.ds(sid * NSLICE, NSLICE)])
    plsc.subcore_barrier()
    pltpu.sync_copy(src_hbm.at[pl.ds(wid * CHUNKS_PER_TILE, CHUNKS_PER_TILE)], sidx)
    pltpu.sync_copy(dst_hbm.at[pl.ds(wid * CHUNKS_PER_TILE, CHUNKS_PER_TILE)], didx)

    def group_body(g, carry):
        j0 = g * GROUP
        gd = []
        for i in range(GROUP):
            gd.append(pltpu.async_copy(
                g_hbm.at[sidx.at[j0 + i]], rows.at[i], gsem))
        for d in gd:
            d.wait()
        sd = []
        for i in range(GROUP):
            sd.append(pltpu.async_copy(
                rows.at[i], acc_sh.at[didx.at[j0 + i]], ssem, add=True))
        for d in sd:
            d.wait()
        return carry

    lax.fori_loop(0, CHUNKS_PER_TILE // GROUP, group_body, 0)
    plsc.subcore_barrier()
    pltpu.sync_copy(acc_sh.at[pl.ds(sid * NSLICE, NSLICE)],
                    acc_hbm.at[cid, pl.ds(sid * NSLICE, NSLICE)])


_FBLK = 1000


def _fin_body(acc_ref, g_ref, dinv_ref, b_ref, out_ref):
    out_ref[...] = (dinv_ref[...] * (acc_ref[0] + acc_ref[1] - g_ref[...])
                    + b_ref[...])


_fin_call = pl.pallas_call(
    _fin_body,
    grid=(N // _FBLK,),
    in_specs=[
        pl.BlockSpec((NC, _FBLK, F), lambda i: (0, i, 0)),
        pl.BlockSpec((_FBLK, F), lambda i: (i, 0)),
        pl.BlockSpec((_FBLK, F), lambda i: (i, 0)),
        pl.BlockSpec((1, F), lambda i: (0, 0)),
    ],
    out_specs=pl.BlockSpec((_FBLK, F), lambda i: (i, 0)),
    out_shape=jax.ShapeDtypeStruct((N, F), jnp.float32),
)


@functools.partial(
    pl.kernel,
    out_type=jax.ShapeDtypeStruct((B * NF, F), jnp.float32),
    mesh=_MESH,
    compiler_params=pltpu.CompilerParams(use_tc_tiling_on_sc=False),
    scratch_types=[
        pltpu.VMEM((XCH_PER_TILE, XCHUNK), jnp.int32),
        pltpu.VMEM((XROWS, F), jnp.float32),
        pltpu.SemaphoreType.DMA,
    ],
)
def _gather_kernel(out_hbm, x_hbm, y_hbm, xidx, rows, sem):
    cid = lax.axis_index("c")
    sid = lax.axis_index("s")
    wid = cid * NS + sid
    pltpu.sync_copy(x_hbm.at[wid], xidx)
    for h in range(2):
        descs = []
        for i in range(XCH_PER_TILE // 2):
            j = h * (XCH_PER_TILE // 2) + i
            descs.append(pltpu.async_copy(
                out_hbm.at[xidx.at[j]], rows.at[pl.ds(j * XCHUNK, XCHUNK)],
                sem))
        for d in descs:
            d.wait()
    pltpu.sync_copy(rows, y_hbm.at[pl.ds(wid * XROWS, XROWS)])


@jax.jit
def _run(features, train_mat, W, b, x):
    srcr = train_mat[0].reshape(E // CHUNK, CHUNK)
    dstr = train_mat[1].reshape(E // CHUNK, CHUNK)
    featp = jnp.pad(features, ((0, NPAD - N), (0, 0)))
    deg_flat = _deg_kernel(dstr)
    degT = deg_flat.reshape(NC, NPAD).T
    g, dinv16 = _mm_call(degT, featp, W)
    acc_parts = _agg_kernel(g, srcr, dstr)
    out = _fin_call(acc_parts, g, dinv16, b.reshape(1, F))
    xr = x.reshape(NW, XCH_PER_TILE, XCHUNK)
    y = _gather_kernel(out, xr)
    return y.reshape(B, NF, F)


def kernel(features, train_mat, W, b, x):
    return _run(features, train_mat, W, b, x)


# speedup vs baseline: 32.1066x; 1.0801x over previous
import functools

import jax
import jax.numpy as jnp
from jax import lax
from jax.experimental import pallas as pl
from jax.experimental.pallas import tpu as pltpu
from jax.experimental.pallas import tpu_sc as plsc

N = 10000
D = 128
F = 16
E = 320000
B = 4096
NF = 26
NC, NS = 2, 16
NW = NC * NS
NPAD = 10240
NSLICE = NPAD // NS
NSLICE32 = NPAD // NW
CHUNK = 125
ECHUNKS = E // CHUNK
CPT_HALF = E // NW // CHUNK
CPT_FULL = E // NS // CHUNK
GROUP = 10
XCHUNK = 128
XCH_PER_TILE = B * NF // NW // XCHUNK
XROWS = B * NF // NW

_MESH = plsc.VectorSubcoreMesh(
    core_axis_name="c", subcore_axis_name="s", num_cores=NC, num_subcores=NS)
_SC_PARAMS = pltpu.CompilerParams(
    use_tc_tiling_on_sc=False, needs_layout_passes=False)


def _rsqrt16(x):
    i = plsc.bitcast(x, jnp.int32)
    y = plsc.bitcast(jnp.int32(0x5F3759DF) - (i >> 1), jnp.float32)
    for _ in range(3):
        y = y * (1.5 - 0.5 * x * y * y)
    return y


_MMBLK = 1024


def _mm_body(feat_ref, w_ref, h_ref):
    h_ref[...] = jnp.dot(feat_ref[...], w_ref[...],
                         preferred_element_type=jnp.float32)


_mm_call = pl.pallas_call(
    _mm_body,
    grid=(NPAD // _MMBLK,),
    in_specs=[
        pl.BlockSpec((_MMBLK, D), lambda i: (i, 0)),
        pl.BlockSpec((D, F), lambda i: (0, 0)),
    ],
    out_specs=pl.BlockSpec((_MMBLK, F), lambda i: (i, 0)),
    out_shape=jax.ShapeDtypeStruct((NPAD, F), jnp.float32),
)


@functools.partial(
    pl.kernel,
    out_type=jax.ShapeDtypeStruct((NC * NPAD,), jnp.float32),
    mesh=_MESH,
    compiler_params=_SC_PARAMS,
    scratch_types=[
        pltpu.VMEM((CPT_FULL, CHUNK), jnp.int32),
        pltpu.VMEM((128,), jnp.float32),
        pltpu.VMEM((NSLICE,), jnp.float32),
        pltpu.VMEM_SHARED((NPAD,), jnp.float32),
        pltpu.SemaphoreType.DMA,
    ],
)
def _deg_kernel(dst_hbm, deg_hbm, didx, ones, zbuf, deg_sh, sem):
    cid = lax.axis_index("c")
    sid = lax.axis_index("s")
    for i in range(128 // F):
        ones[pl.ds(i * F, F)] = jnp.ones((F,), jnp.float32)
    for i in range(NSLICE // F):
        zbuf[pl.ds(i * F, F)] = jnp.zeros((F,), jnp.float32)
    pltpu.sync_copy(zbuf, deg_sh.at[pl.ds(sid * NSLICE, NSLICE)])
    plsc.subcore_barrier()
    pltpu.sync_copy(dst_hbm.at[pl.ds(sid * CPT_FULL, CPT_FULL)], didx)

    def group_body(gi, carry):
        j0 = gi * GROUP
        descs = []
        for i in range(GROUP):
            descs.append(pltpu.async_copy(
                ones.at[pl.ds(0, CHUNK)], deg_sh.at[didx.at[j0 + i]], sem,
                add=True))
        for d in descs:
            d.wait()
        return carry

    lax.fori_loop(0, CPT_FULL // GROUP, group_body, 0)
    plsc.subcore_barrier()
    pltpu.sync_copy(deg_sh.at[pl.ds(sid * NSLICE, NSLICE)],
                    deg_hbm.at[pl.ds(cid * NPAD + sid * NSLICE, NSLICE)])


@functools.partial(
    pl.kernel,
    out_type=(
        jax.ShapeDtypeStruct((NPAD, F), jnp.float32),
        jax.ShapeDtypeStruct((NPAD,), jnp.float32),
        jax.ShapeDtypeStruct((NC, NPAD, F), jnp.float32),
    ),
    mesh=_MESH,
    compiler_params=_SC_PARAMS,
    scratch_types=[
        pltpu.VMEM((NSLICE,), jnp.float32),
        pltpu.VMEM((NSLICE,), jnp.float32),
        pltpu.VMEM((NSLICE, F), jnp.float32),
        pltpu.VMEM((CPT_HALF, CHUNK), jnp.int32),
        pltpu.VMEM((CPT_HALF, CHUNK), jnp.int32),
        pltpu.VMEM((GROUP, CHUNK, F), jnp.float32),
        pltpu.VMEM_SHARED((NPAD, F), jnp.float32),
        pltpu.SemaphoreType.DMA,
        pltpu.SemaphoreType.DMA,
    ],
)
def _agg_kernel(deg_hbm, h_hbm, src_hbm, dst_hbm, g_hbm, dinv_hbm, acc_hbm,
                degb, dinvb, hb, sidx, didx, rows, acc_sh, gsem, ssem):
    cid = lax.axis_index("c")
    sid = lax.axis_index("s")
    wid = cid * NS + sid
    base = sid * NSLICE
    pltpu.sync_copy(deg_hbm.at[pl.ds(cid * NPAD + base, NSLICE)], degb)

    def rsqrt_body(k, carry):
        v = degb[pl.ds(k * F, F)] + 1.0
        dinvb[pl.ds(k * F, F)] = _rsqrt16(v)
        return carry

    lax.fori_loop(0, NSLICE // F, rsqrt_body, 0)
    pltpu.sync_copy(dinvb, dinv_hbm.at[pl.ds(base, NSLICE)])
    pltpu.sync_copy(h_hbm.at[pl.ds(base, NSLICE)], hb)

    def scale_body(k, carry):
        dv = dinvb[pl.ds(k * F, F)]
        for l in range(F):
            r = k * F + l
            hb[r, :] = hb[r, :] * dv[l]
        return carry

    lax.fori_loop(0, NSLICE // F, scale_body, 0)
    pltpu.sync_copy(hb, g_hbm.at[pl.ds(base, NSLICE)])
    pltpu.sync_copy(hb, acc_sh.at[pl.ds(base, NSLICE)])
    plsc.subcore_barrier()
    pltpu.sync_copy(src_hbm.at[pl.ds(wid * CPT_HALF, CPT_HALF)], sidx)
    pltpu.sync_copy(dst_hbm.at[pl.ds(wid * CPT_HALF, CPT_HALF)], didx)

    def group_body(gi, carry):
        j0 = gi * GROUP
        gd = []
        for i in range(GROUP):
            gd.append(pltpu.async_copy(
                g_hbm.at[sidx.at[j0 + i]], rows.at[i], gsem))
        for d in gd:
            d.wait()
        sd = []
        for i in range(GROUP):
            sd.append(pltpu.async_copy(
                rows.at[i], acc_sh.at[didx.at[j0 + i]], ssem, add=True))
        for d in sd:
            d.wait()
        return carry

    lax.fori_loop(0, CPT_HALF // GROUP, group_body, 0)
    plsc.subcore_barrier()
    pltpu.sync_copy(acc_sh.at[pl.ds(base, NSLICE)],
                    acc_hbm.at[cid, pl.ds(base, NSLICE)])


@functools.partial(
    pl.kernel,
    out_type=jax.ShapeDtypeStruct((NPAD, F), jnp.float32),
    mesh=_MESH,
    compiler_params=_SC_PARAMS,
    scratch_types=[
        pltpu.VMEM((NSLICE32, F), jnp.float32),
        pltpu.VMEM((NSLICE32, F), jnp.float32),
        pltpu.VMEM((NSLICE32, F), jnp.float32),
        pltpu.VMEM((NSLICE32,), jnp.float32),
        pltpu.VMEM((F,), jnp.float32),
    ],
)
def _fin_kernel(acc_hbm, g_hbm, dinv_hbm, b_hbm, out_hbm,
                a0, a1, gb, dinvb, bb):
    cid = lax.axis_index("c")
    sid = lax.axis_index("s")
    wid = cid * NS + sid
    base = wid * NSLICE32
    pltpu.sync_copy(acc_hbm.at[0, pl.ds(base, NSLICE32)], a0)
    pltpu.sync_copy(acc_hbm.at[1, pl.ds(base, NSLICE32)], a1)
    pltpu.sync_copy(g_hbm.at[pl.ds(base, NSLICE32)], gb)
    pltpu.sync_copy(dinv_hbm.at[pl.ds(base, NSLICE32)], dinvb)
    pltpu.sync_copy(b_hbm, bb)
    bv = bb[...]

    def row_body(k, carry):
        dv = dinvb[pl.ds(k * F, F)]
        for l in range(F):
            r = k * F + l
            gb[r, :] = (a0[r, :] + a1[r, :] - gb[r, :]) * dv[l] + bv
        return carry

    lax.fori_loop(0, NSLICE32 // F, row_body, 0)
    pltpu.sync_copy(gb, out_hbm.at[pl.ds(base, NSLICE32)])


@functools.partial(
    pl.kernel,
    out_type=jax.ShapeDtypeStruct((B * NF, F), jnp.float32),
    mesh=_MESH,
    compiler_params=_SC_PARAMS,
    scratch_types=[
        pltpu.VMEM((XCH_PER_TILE, XCHUNK), jnp.int32),
        pltpu.VMEM((XROWS, F), jnp.float32),
        pltpu.SemaphoreType.DMA,
    ],
)
def _gather_kernel(out_hbm, x_hbm, y_hbm, xidx, rows, sem):
    cid = lax.axis_index("c")
    sid = lax.axis_index("s")
    wid = cid * NS + sid
    pltpu.sync_copy(x_hbm.at[wid], xidx)
    for h in range(2):
        descs = []
        for i in range(XCH_PER_TILE // 2):
            j = h * (XCH_PER_TILE // 2) + i
            descs.append(pltpu.async_copy(
                out_hbm.at[xidx.at[j]], rows.at[pl.ds(j * XCHUNK, XCHUNK)],
                sem))
        for d in descs:
            d.wait()
    pltpu.sync_copy(rows, y_hbm.at[pl.ds(wid * XROWS, XROWS)])


@jax.jit
def _run(features, train_mat, W, b, x):
    srcr = train_mat[0].reshape(ECHUNKS, CHUNK)
    dstr = train_mat[1].reshape(ECHUNKS, CHUNK)
    featp = jnp.pad(features, ((0, NPAD - N), (0, 0)))
    h = _mm_call(featp, W)
    deg_flat = _deg_kernel(dstr)
    g, dinv, acc_parts = _agg_kernel(deg_flat, h, srcr, dstr)
    out = _fin_kernel(acc_parts, g, dinv, b)
    xr = x.reshape(NW, XCH_PER_TILE, XCHUNK)
    y = _gather_kernel(out, xr)
    return y.reshape(B, NF, F)


def kernel(features, train_mat, W, b, x):
    return _run(features, train_mat, W, b, x)


# speedup vs baseline: 49.3003x; 1.5355x over previous
import functools

import jax
import jax.numpy as jnp
from jax import lax
from jax.experimental import pallas as pl
from jax.experimental.pallas import tpu as pltpu
from jax.experimental.pallas import tpu_sc as plsc

N = 10000
D = 128
F = 16
E = 320000
B = 4096
NF = 26
NC, NS = 2, 16
NW = NC * NS
NPAD = 10240
NSLICE = NPAD // NS
NSLICE32 = NPAD // NW
CHUNK = 125
ECHUNKS = E // CHUNK
CPT_HALF = E // NW // CHUNK
CPT_FULL = E // NS // CHUNK
GROUP = 10
XCHUNK = 128
XCH_PER_TILE = B * NF // NW // XCHUNK
XROWS = B * NF // NW

_MESH = plsc.VectorSubcoreMesh(
    core_axis_name="c", subcore_axis_name="s", num_cores=NC, num_subcores=NS)
_SC_PARAMS = pltpu.CompilerParams(
    use_tc_tiling_on_sc=False, needs_layout_passes=False)


def _rsqrt16(x):
    i = plsc.bitcast(x, jnp.int32)
    y = plsc.bitcast(jnp.int32(0x5F3759DF) - (i >> 1), jnp.float32)
    for _ in range(3):
        y = y * (1.5 - 0.5 * x * y * y)
    return y


_MMBLK = 1024


def _mm_body(feat_ref, w_ref, h_ref):
    h_ref[...] = jnp.dot(feat_ref[...], w_ref[...],
                         preferred_element_type=jnp.float32)


_mm_call = pl.pallas_call(
    _mm_body,
    grid=(NPAD // _MMBLK,),
    in_specs=[
        pl.BlockSpec((_MMBLK, D), lambda i: (i, 0)),
        pl.BlockSpec((D, F), lambda i: (0, 0)),
    ],
    out_specs=pl.BlockSpec((_MMBLK, F), lambda i: (i, 0)),
    out_shape=jax.ShapeDtypeStruct((NPAD, F), jnp.float32),
)


@functools.partial(
    pl.kernel,
    out_type=jax.ShapeDtypeStruct((NC * NPAD,), jnp.float32),
    mesh=_MESH,
    compiler_params=_SC_PARAMS,
    scratch_types=[
        pltpu.VMEM((CPT_FULL, CHUNK), jnp.int32),
        pltpu.VMEM((128,), jnp.float32),
        pltpu.VMEM((NSLICE,), jnp.float32),
        pltpu.VMEM_SHARED((NPAD,), jnp.float32),
        pltpu.SemaphoreType.DMA,
    ],
)
def _deg_kernel(dst_hbm, deg_hbm, didx, ones, zbuf, deg_sh, sem):
    cid = lax.axis_index("c")
    sid = lax.axis_index("s")
    for i in range(128 // F):
        ones[pl.ds(i * F, F)] = jnp.ones((F,), jnp.float32)
    for i in range(NSLICE // F):
        zbuf[pl.ds(i * F, F)] = jnp.zeros((F,), jnp.float32)
    pltpu.sync_copy(zbuf, deg_sh.at[pl.ds(sid * NSLICE, NSLICE)])
    plsc.subcore_barrier()
    pltpu.sync_copy(dst_hbm.at[pl.ds(sid * CPT_FULL, CPT_FULL)], didx)

    def group_body(gi, carry):
        j0 = gi * GROUP
        descs = []
        for i in range(GROUP):
            descs.append(pltpu.async_copy(
                ones.at[pl.ds(0, CHUNK)], deg_sh.at[didx.at[j0 + i]], sem,
                add=True))
        for d in descs:
            d.wait()
        return carry

    lax.fori_loop(0, CPT_FULL // GROUP, group_body, 0)
    plsc.subcore_barrier()
    pltpu.sync_copy(deg_sh.at[pl.ds(sid * NSLICE, NSLICE)],
                    deg_hbm.at[pl.ds(cid * NPAD + sid * NSLICE, NSLICE)])


@functools.partial(
    pl.kernel,
    out_type=(
        jax.ShapeDtypeStruct((NPAD, F), jnp.float32),
        jax.ShapeDtypeStruct((NPAD,), jnp.float32),
        jax.ShapeDtypeStruct((NC, NPAD, F), jnp.float32),
    ),
    mesh=_MESH,
    compiler_params=_SC_PARAMS,
    scratch_types=[
        pltpu.VMEM((NSLICE,), jnp.float32),
        pltpu.VMEM((NSLICE,), jnp.float32),
        pltpu.VMEM((NSLICE, F), jnp.float32),
        pltpu.VMEM((CPT_HALF, CHUNK), jnp.int32),
        pltpu.VMEM((CPT_HALF, CHUNK), jnp.int32),
        pltpu.VMEM((GROUP, CHUNK, F), jnp.float32),
        pltpu.VMEM_SHARED((NPAD, F), jnp.float32),
        pltpu.SemaphoreType.DMA,
        pltpu.SemaphoreType.DMA,
    ],
)
def _agg_kernel(deg_hbm, h_hbm, src_hbm, dst_hbm, g_hbm, dinv_hbm, acc_hbm,
                degb, dinvb, hb, sidx, didx, rows, acc_sh, gsem, ssem):
    cid = lax.axis_index("c")
    sid = lax.axis_index("s")
    wid = cid * NS + sid
    base = sid * NSLICE
    pltpu.sync_copy(deg_hbm.at[pl.ds(cid * NPAD + base, NSLICE)], degb)

    def rsqrt_body(k, carry):
        v = degb[pl.ds(k * F, F)] + 1.0
        dinvb[pl.ds(k * F, F)] = _rsqrt16(v)
        return carry

    lax.fori_loop(0, NSLICE // F, rsqrt_body, 0)
    pltpu.sync_copy(dinvb, dinv_hbm.at[pl.ds(base, NSLICE)])
    pltpu.sync_copy(h_hbm.at[pl.ds(base, NSLICE)], hb)

    def scale_body(k, carry):
        dv = dinvb[pl.ds(k * F, F)]
        for l in range(F):
            r = k * F + l
            hb[r, :] = hb[r, :] * dv[l]
        return carry

    lax.fori_loop(0, NSLICE // F, scale_body, 0)
    pltpu.sync_copy(hb, g_hbm.at[pl.ds(base, NSLICE)])
    pltpu.sync_copy(hb, acc_sh.at[pl.ds(base, NSLICE)])
    plsc.subcore_barrier()
    pltpu.sync_copy(src_hbm.at[pl.ds(wid * CPT_HALF, CPT_HALF)], sidx)
    pltpu.sync_copy(dst_hbm.at[pl.ds(wid * CPT_HALF, CPT_HALF)], didx)

    def group_body(gi, carry):
        j0 = gi * GROUP
        gd = []
        for i in range(GROUP):
            gd.append(pltpu.async_copy(
                g_hbm.at[sidx.at[j0 + i]], rows.at[i], gsem))
        for d in gd:
            d.wait()
        sd = []
        for i in range(GROUP):
            sd.append(pltpu.async_copy(
                rows.at[i], acc_sh.at[didx.at[j0 + i]], ssem, add=True))
        for d in sd:
            d.wait()
        return carry

    lax.fori_loop(0, CPT_HALF // GROUP, group_body, 0)
    plsc.subcore_barrier()
    pltpu.sync_copy(acc_sh.at[pl.ds(base, NSLICE)],
                    acc_hbm.at[cid, pl.ds(base, NSLICE)])


@functools.partial(
    pl.kernel,
    out_type=jax.ShapeDtypeStruct((NPAD, F), jnp.float32),
    mesh=_MESH,
    compiler_params=_SC_PARAMS,
    scratch_types=[
        pltpu.VMEM((NSLICE32, F), jnp.float32),
        pltpu.VMEM((NSLICE32, F), jnp.float32),
        pltpu.VMEM((NSLICE32, F), jnp.float32),
        pltpu.VMEM((NSLICE32,), jnp.float32),
        pltpu.VMEM((F,), jnp.float32),
    ],
)
def _fin_kernel(acc_hbm, g_hbm, dinv_hbm, b_hbm, out_hbm,
                a0, a1, gb, dinvb, bb):
    cid = lax.axis_index("c")
    sid = lax.axis_index("s")
    wid = cid * NS + sid
    base = wid * NSLICE32
    pltpu.sync_copy(acc_hbm.at[0, pl.ds(base, NSLICE32)], a0)
    pltpu.sync_copy(acc_hbm.at[1, pl.ds(base, NSLICE32)], a1)
    pltpu.sync_copy(g_hbm.at[pl.ds(base, NSLICE32)], gb)
    pltpu.sync_copy(dinv_hbm.at[pl.ds(base, NSLICE32)], dinvb)
    pltpu.sync_copy(b_hbm, bb)
    bv = bb[...]

    def row_body(k, carry):
        dv = dinvb[pl.ds(k * F, F)]
        for l in range(F):
            r = k * F + l
            gb[r, :] = (a0[r, :] + a1[r, :] - gb[r, :]) * dv[l] + bv
        return carry

    lax.fori_loop(0, NSLICE32 // F, row_body, 0)
    pltpu.sync_copy(gb, out_hbm.at[pl.ds(base, NSLICE32)])


@functools.partial(
    pl.kernel,
    out_type=jax.ShapeDtypeStruct((NF, F, B), jnp.float32),
    mesh=_MESH,
    compiler_params=_SC_PARAMS,
    scratch_types=[
        pltpu.VMEM((NF, XCHUNK), jnp.int32),
        pltpu.VMEM((NF, XCHUNK, F), jnp.float32),
        pltpu.VMEM((NF, F, XCHUNK), jnp.float32),
        pltpu.SemaphoreType.DMA,
        pltpu.SemaphoreType.DMA,
    ],
)
def _gather_kernel(out_hbm, xt_hbm, y_hbm, xidx, rows, slabs, gsem, wsem):
    cid = lax.axis_index("c")
    sid = lax.axis_index("s")
    wid = cid * NS + sid
    ibase = wid * XCHUNK
    pltpu.sync_copy(xt_hbm.at[:, pl.ds(ibase, XCHUNK)], xidx)
    lane = lax.iota(jnp.int32, F)

    def transpose_field(jj, carry):
        jv = jnp.full((F,), jj, jnp.int32)
        for r in range(XCHUNK):
            v = rows[jj, r, :]
            plsc.store_scatter(
                slabs, [jv, lane, jnp.full((F,), r, jnp.int32)], v)
        return carry

    half = NF // 2
    descs = []
    for j in range(half):
        descs.append(pltpu.async_copy(
            out_hbm.at[xidx.at[j]], rows.at[j], gsem))
    for d in descs:
        d.wait()
    descs = []
    for j in range(half, NF):
        descs.append(pltpu.async_copy(
            out_hbm.at[xidx.at[j]], rows.at[j], gsem))
    lax.fori_loop(0, half, transpose_field, 0)
    for d in descs:
        d.wait()
    lax.fori_loop(half, NF, transpose_field, 0)
    descs = []
    for j in range(NF):
        descs.append(pltpu.async_copy(
            slabs.at[j], y_hbm.at[j, :, pl.ds(ibase, XCHUNK)], wsem))
    for d in descs:
        d.wait()


@jax.jit
def _run(features, train_mat, W, b, x):
    srcr = train_mat[0].reshape(ECHUNKS, CHUNK)
    dstr = train_mat[1].reshape(ECHUNKS, CHUNK)
    featp = jnp.pad(features, ((0, NPAD - N), (0, 0)))
    h = _mm_call(featp, W)
    deg_flat = _deg_kernel(dstr)
    g, dinv, acc_parts = _agg_kernel(deg_flat, h, srcr, dstr)
    out = _fin_kernel(acc_parts, g, dinv, b)
    y = _gather_kernel(out, x.T)
    return jnp.transpose(y, (2, 0, 1))


def kernel(features, train_mat, W, b, x):
    return _run(features, train_mat, W, b, x)
